# baseline (speedup 1.0000x reference)
import jax
import jax.numpy as jnp
from jax.experimental import pallas as pl
from jax.experimental.pallas import tpu as pltpu

_TT = 512
_U = 8
_C = 2


def _sigmoid(x):
    return jax.nn.sigmoid(x)


def kernel(x, W_ih, W_hh, b_ih, b_hh):
    B, T, IN = x.shape
    H = W_hh.shape[1]
    G4 = 4 * H
    NT = T // _TT
    NG = _TT // _U
    BC = B // _C

    bias = b_ih + b_hh
    b_hi = bias.astype(jnp.bfloat16).astype(jnp.float32)
    Waug = jnp.zeros((2 * H, G4), jnp.float32)
    Waug = Waug.at[:H, :].set(W_hh.T)
    Waug = Waug.at[H, :].set(W_ih[:, 0])
    Waug = Waug.at[H + 1, :].set(b_hi)
    Waug = Waug.at[H + 2, :].set(bias - b_hi)
    lane = jnp.arange(H)
    misc = jnp.stack(
        [
            (lane == 0).astype(jnp.float32),
            ((lane == 1) | (lane == 2)).astype(jnp.float32),
        ]
    )
    xg = jnp.transpose(x[:, :, 0].reshape(B, T // _U, _U), (1, 0, 2))

    def body(x_ref, w_ref, misc_ref, hist_ref, sfin_ref, h_s, c_s):
        j = pl.program_id(0)
        e0 = misc_ref[0:1, :]
        e1 = misc_ref[1:2, :]

        @pl.when(j == 0)
        def _():
            h_s[...] = jnp.zeros_like(h_s)
            c_s[...] = jnp.zeros_like(c_s)

        Wv = w_ref[...].astype(jnp.bfloat16)
        pltpu.matmul_push_rhs(Wv[:, : 2 * H], 0, 0)
        pltpu.matmul_push_rhs(Wv[:, 2 * H :], 0, 1)
        zlat = jnp.zeros((8, 2 * H), jnp.float32)
        pltpu.matmul_acc_lhs(32, zlat, 0, load_staged_rhs=0)
        pltpu.matmul_acc_lhs(32, zlat, 1, load_staged_rhs=0)

        def group(gi, carry):
            hs, cs = carry
            t8 = gi * _U
            xs_grp = x_ref[pl.ds(gi, 1), :, :].reshape(B, _U)
            xpads = [xs_grp[:, u : u + 1] * e0 + e1 for u in range(_U)]
            hlists = [[] for _ in range(_C)]
            for u in range(_U):
                xpad = xpads[u]
                for cc in range(_C):
                    b0 = cc * BC
                    haug = jnp.concatenate([hs[cc], xpad[b0 : b0 + BC, :]], axis=1)
                    pltpu.matmul_acc_lhs(cc * 4, haug, 0, load_staged_rhs=None)
                    pltpu.matmul_acc_lhs(cc * 4, haug, 1, load_staged_rhs=None)
                new_hs, new_cs = [], []
                for cc in range(_C):
                    glo = pltpu.matmul_pop(cc * 4, (BC, 2 * H), jnp.float32, 0)
                    ghi = pltpu.matmul_pop(cc * 4, (BC, 2 * H), jnp.float32, 1)
                    i_g = _sigmoid(glo[:, :H])
                    f_g = _sigmoid(glo[:, H:])
                    g_g = jnp.tanh(ghi[:, :H])
                    o_g = _sigmoid(ghi[:, H:])
                    c_new = f_g * cs[cc] + i_g * g_g
                    h_new = o_g * jnp.tanh(c_new)
                    h = 0.5 * (hs[cc] + h_new)
                    c = 0.5 * (cs[cc] + c_new)
                    hlists[cc].append(h)
                    new_hs.append(h)
                    new_cs.append(c)
                hs, cs = tuple(new_hs), tuple(new_cs)
            tt0 = pl.multiple_of(t8, _U)
            for cc in range(_C):
                b0 = cc * BC
                hist_ref[b0 : b0 + BC, pl.ds(tt0, _U), :] = jnp.stack(
                    hlists[cc], axis=1
                )
            return (hs, cs)

        h0 = tuple(h_s[cc * BC : (cc + 1) * BC, :] for cc in range(_C))
        c0 = tuple(c_s[cc * BC : (cc + 1) * BC, :] for cc in range(_C))
        hs, cs = jax.lax.fori_loop(0, NG, group, (h0, c0))
        for cc in range(_C):
            h_s[cc * BC : (cc + 1) * BC, :] = hs[cc]
            c_s[cc * BC : (cc + 1) * BC, :] = cs[cc]

        @pl.when(j == NT - 1)
        def _():
            for cc in range(_C):
                b0 = cc * BC
                sfin_ref[b0 : b0 + BC, :H] = hs[cc]
                sfin_ref[b0 : b0 + BC, H:] = cs[cc]

    hist, sfin = pl.pallas_call(
        body,
        grid=(NT,),
        in_specs=[
            pl.BlockSpec((_TT // _U, B, _U), lambda j: (j, 0, 0)),
            pl.BlockSpec((2 * H, G4), lambda j: (0, 0)),
            pl.BlockSpec((2, H), lambda j: (0, 0)),
        ],
        out_specs=[
            pl.BlockSpec((B, _TT, H), lambda j: (0, j, 0)),
            pl.BlockSpec((B, 2 * H), lambda j: (0, 0)),
        ],
        out_shape=[
            jax.ShapeDtypeStruct((B, T, H), jnp.float32),
            jax.ShapeDtypeStruct((B, 2 * H), jnp.float32),
        ],
        scratch_shapes=[
            pltpu.VMEM((B, H), jnp.float32),
            pltpu.VMEM((B, H), jnp.float32),
        ],
        compiler_params=pltpu.CompilerParams(
            dimension_semantics=("arbitrary",),
        ),
    )(xg, Waug, misc)

    return hist, sfin


# speedup vs baseline: 1.0750x; 1.0750x over previous
import jax
import jax.numpy as jnp
from jax.experimental import pallas as pl
from jax.experimental.pallas import tpu as pltpu

_TT = 512
_U = 16
_C = 2


def _sigmoid(x):
    return 0.5 * jnp.tanh(0.5 * x) + 0.5


def kernel(x, W_ih, W_hh, b_ih, b_hh):
    B, T, IN = x.shape
    H = W_hh.shape[1]
    G4 = 4 * H
    NT = T // _TT
    NG = _TT // _U
    BC = B // _C

    bias = b_ih + b_hh
    b_hi = bias.astype(jnp.bfloat16).astype(jnp.float32)
    Waug = jnp.zeros((2 * H, G4), jnp.float32)
    Waug = Waug.at[:H, :].set(W_hh.T)
    Waug = Waug.at[H, :].set(W_ih[:, 0])
    Waug = Waug.at[H + 1, :].set(b_hi)
    Waug = Waug.at[H + 2, :].set(bias - b_hi)
    lane = jnp.arange(H)
    misc = jnp.stack(
        [
            (lane == 0).astype(jnp.float32),
            ((lane == 1) | (lane == 2)).astype(jnp.float32),
        ]
    )
    xg = jnp.transpose(x[:, :, 0].reshape(B, T // _U, _U), (1, 0, 2))

    def body(x_ref, w_ref, misc_ref, hist_ref, sfin_ref, h_s, c_s):
        j = pl.program_id(0)
        e0 = misc_ref[0:1, :]
        e1 = misc_ref[1:2, :]

        @pl.when(j == 0)
        def _():
            h_s[...] = jnp.zeros_like(h_s)
            c_s[...] = jnp.zeros_like(c_s)

        Wv = w_ref[...].astype(jnp.bfloat16)
        pltpu.matmul_push_rhs(Wv[:, : 2 * H], 0, 0)
        pltpu.matmul_push_rhs(Wv[:, 2 * H :], 0, 1)
        zlat = jnp.zeros((8, 2 * H), jnp.float32)
        pltpu.matmul_acc_lhs(32, zlat, 0, load_staged_rhs=0)
        pltpu.matmul_acc_lhs(32, zlat, 1, load_staged_rhs=0)

        def group(gi, carry):
            hs, cs = carry
            t8 = gi * _U
            xs_grp = x_ref[pl.ds(gi, 1), :, :].reshape(B, _U)
            xpads = [xs_grp[:, u : u + 1] * e0 + e1 for u in range(_U)]
            hlists = [[] for _ in range(_C)]
            for u in range(_U):
                xpad = xpads[u]
                for cc in range(_C):
                    b0 = cc * BC
                    haug = jnp.concatenate([hs[cc], xpad[b0 : b0 + BC, :]], axis=1)
                    pltpu.matmul_acc_lhs(cc * 4, haug, 0, load_staged_rhs=None)
                    pltpu.matmul_acc_lhs(cc * 4, haug, 1, load_staged_rhs=None)
                new_hs, new_cs = [], []
                for cc in range(_C):
                    glo = pltpu.matmul_pop(cc * 4, (BC, 2 * H), jnp.float32, 0)
                    ghi = pltpu.matmul_pop(cc * 4, (BC, 2 * H), jnp.float32, 1)
                    i_g = _sigmoid(glo[:, :H])
                    f_g = _sigmoid(glo[:, H:])
                    g_g = jnp.tanh(ghi[:, :H])
                    o_g = _sigmoid(ghi[:, H:])
                    c_new = f_g * cs[cc] + i_g * g_g
                    h_new = o_g * jnp.tanh(c_new)
                    h = 0.5 * (hs[cc] + h_new)
                    c = 0.5 * (cs[cc] + c_new)
                    hlists[cc].append(h)
                    new_hs.append(h)
                    new_cs.append(c)
                hs, cs = tuple(new_hs), tuple(new_cs)
            tt0 = pl.multiple_of(t8, _U)
            for cc in range(_C):
                b0 = cc * BC
                hist_ref[b0 : b0 + BC, pl.ds(tt0, _U), :] = jnp.stack(
                    hlists[cc], axis=1
                )
            return (hs, cs)

        h0 = tuple(h_s[cc * BC : (cc + 1) * BC, :] for cc in range(_C))
        c0 = tuple(c_s[cc * BC : (cc + 1) * BC, :] for cc in range(_C))
        hs, cs = jax.lax.fori_loop(0, NG, group, (h0, c0))
        for cc in range(_C):
            h_s[cc * BC : (cc + 1) * BC, :] = hs[cc]
            c_s[cc * BC : (cc + 1) * BC, :] = cs[cc]

        @pl.when(j == NT - 1)
        def _():
            for cc in range(_C):
                b0 = cc * BC
                sfin_ref[b0 : b0 + BC, :H] = hs[cc]
                sfin_ref[b0 : b0 + BC, H:] = cs[cc]

    hist, sfin = pl.pallas_call(
        body,
        grid=(NT,),
        in_specs=[
            pl.BlockSpec((_TT // _U, B, _U), lambda j: (j, 0, 0)),
            pl.BlockSpec((2 * H, G4), lambda j: (0, 0)),
            pl.BlockSpec((2, H), lambda j: (0, 0)),
        ],
        out_specs=[
            pl.BlockSpec((B, _TT, H), lambda j: (0, j, 0)),
            pl.BlockSpec((B, 2 * H), lambda j: (0, 0)),
        ],
        out_shape=[
            jax.ShapeDtypeStruct((B, T, H), jnp.float32),
            jax.ShapeDtypeStruct((B, 2 * H), jnp.float32),
        ],
        scratch_shapes=[
            pltpu.VMEM((B, H), jnp.float32),
            pltpu.VMEM((B, H), jnp.float32),
        ],
        compiler_params=pltpu.CompilerParams(
            dimension_semantics=("arbitrary",),
        ),
    )(xg, Waug, misc)

    return hist, sfin


# speedup vs baseline: 1.1013x; 1.0244x over previous
import jax
import jax.numpy as jnp
from jax.experimental import pallas as pl
from jax.experimental.pallas import tpu as pltpu

_TT = 512
_U = 32
_C = 2


def _sigmoid(x):
    return 0.5 * jnp.tanh(0.5 * x) + 0.5


def kernel(x, W_ih, W_hh, b_ih, b_hh):
    B, T, IN = x.shape
    H = W_hh.shape[1]
    G4 = 4 * H
    NT = T // _TT
    NG = _TT // _U
    BC = B // _C

    bias = b_ih + b_hh
    b_hi = bias.astype(jnp.bfloat16).astype(jnp.float32)
    Waug = jnp.zeros((2 * H, G4), jnp.float32)
    Waug = Waug.at[:H, :].set(W_hh.T)
    Waug = Waug.at[H, :].set(W_ih[:, 0])
    Waug = Waug.at[H + 1, :].set(b_hi)
    Waug = Waug.at[H + 2, :].set(bias - b_hi)
    lane = jnp.arange(H)
    misc = jnp.stack(
        [
            (lane == 0).astype(jnp.float32),
            ((lane == 1) | (lane == 2)).astype(jnp.float32),
        ]
    )
    xg = jnp.transpose(x[:, :, 0].reshape(B, T // _U, _U), (1, 0, 2))

    def body(x_ref, w_ref, misc_ref, hist_ref, sfin_ref, h_s, c_s):
        j = pl.program_id(0)
        e0 = misc_ref[0:1, :]
        e1 = misc_ref[1:2, :]

        @pl.when(j == 0)
        def _():
            h_s[...] = jnp.zeros_like(h_s)
            c_s[...] = jnp.zeros_like(c_s)

        Wv = w_ref[...].astype(jnp.bfloat16)
        pltpu.matmul_push_rhs(Wv[:, : 2 * H], 0, 0)
        pltpu.matmul_push_rhs(Wv[:, 2 * H :], 0, 1)
        zlat = jnp.zeros((8, 2 * H), jnp.float32)
        pltpu.matmul_acc_lhs(32, zlat, 0, load_staged_rhs=0)
        pltpu.matmul_acc_lhs(32, zlat, 1, load_staged_rhs=0)

        def group(gi, carry):
            hs, cs = carry
            t8 = gi * _U
            xs_grp = x_ref[pl.ds(gi, 1), :, :].reshape(B, _U)
            xpads = [xs_grp[:, u : u + 1] * e0 + e1 for u in range(_U)]
            hlists = [[] for _ in range(_C)]
            for u in range(_U):
                xpad = xpads[u]
                for cc in range(_C):
                    b0 = cc * BC
                    haug = jnp.concatenate([hs[cc], xpad[b0 : b0 + BC, :]], axis=1)
                    pltpu.matmul_acc_lhs(cc * 4, haug, 0, load_staged_rhs=None)
                    pltpu.matmul_acc_lhs(cc * 4, haug, 1, load_staged_rhs=None)
                new_hs, new_cs = [], []
                for cc in range(_C):
                    glo = pltpu.matmul_pop(cc * 4, (BC, 2 * H), jnp.float32, 0)
                    ghi = pltpu.matmul_pop(cc * 4, (BC, 2 * H), jnp.float32, 1)
                    i_g = _sigmoid(glo[:, :H])
                    f_g = _sigmoid(glo[:, H:])
                    g_g = jnp.tanh(ghi[:, :H])
                    o_g = _sigmoid(ghi[:, H:])
                    c_new = f_g * cs[cc] + i_g * g_g
                    h_new = o_g * jnp.tanh(c_new)
                    h = 0.5 * (hs[cc] + h_new)
                    c = 0.5 * (cs[cc] + c_new)
                    hlists[cc].append(h)
                    new_hs.append(h)
                    new_cs.append(c)
                hs, cs = tuple(new_hs), tuple(new_cs)
            tt0 = pl.multiple_of(t8, _U)
            for cc in range(_C):
                b0 = cc * BC
                hist_ref[b0 : b0 + BC, pl.ds(tt0, _U), :] = jnp.stack(
                    hlists[cc], axis=1
                )
            return (hs, cs)

        h0 = tuple(h_s[cc * BC : (cc + 1) * BC, :] for cc in range(_C))
        c0 = tuple(c_s[cc * BC : (cc + 1) * BC, :] for cc in range(_C))
        hs, cs = jax.lax.fori_loop(0, NG, group, (h0, c0))
        for cc in range(_C):
            h_s[cc * BC : (cc + 1) * BC, :] = hs[cc]
            c_s[cc * BC : (cc + 1) * BC, :] = cs[cc]

        @pl.when(j == NT - 1)
        def _():
            for cc in range(_C):
                b0 = cc * BC
                sfin_ref[b0 : b0 + BC, :H] = hs[cc]
                sfin_ref[b0 : b0 + BC, H:] = cs[cc]

    hist, sfin = pl.pallas_call(
        body,
        grid=(NT,),
        in_specs=[
            pl.BlockSpec((_TT // _U, B, _U), lambda j: (j, 0, 0)),
            pl.BlockSpec((2 * H, G4), lambda j: (0, 0)),
            pl.BlockSpec((2, H), lambda j: (0, 0)),
        ],
        out_specs=[
            pl.BlockSpec((B, _TT, H), lambda j: (0, j, 0)),
            pl.BlockSpec((B, 2 * H), lambda j: (0, 0)),
        ],
        out_shape=[
            jax.ShapeDtypeStruct((B, T, H), jnp.float32),
            jax.ShapeDtypeStruct((B, 2 * H), jnp.float32),
        ],
        scratch_shapes=[
            pltpu.VMEM((B, H), jnp.float32),
            pltpu.VMEM((B, H), jnp.float32),
        ],
        compiler_params=pltpu.CompilerParams(
            dimension_semantics=("arbitrary",),
        ),
    )(xg, Waug, misc)

    return hist, sfin


# speedup vs baseline: 1.1203x; 1.0173x over previous
import jax
import jax.numpy as jnp
from jax.experimental import pallas as pl
from jax.experimental.pallas import tpu as pltpu

_TT = 512
_U = 64
_C = 2


def _sigmoid(x):
    return 0.5 * jnp.tanh(0.5 * x) + 0.5


def kernel(x, W_ih, W_hh, b_ih, b_hh):
    B, T, IN = x.shape
    H = W_hh.shape[1]
    G4 = 4 * H
    NT = T // _TT
    NG = _TT // _U
    BC = B // _C

    bias = b_ih + b_hh
    b_hi = bias.astype(jnp.bfloat16).astype(jnp.float32)
    Waug = jnp.zeros((2 * H, G4), jnp.float32)
    Waug = Waug.at[:H, :].set(W_hh.T)
    Waug = Waug.at[H, :].set(W_ih[:, 0])
    Waug = Waug.at[H + 1, :].set(b_hi)
    Waug = Waug.at[H + 2, :].set(bias - b_hi)
    lane = jnp.arange(H)
    misc = jnp.stack(
        [
            (lane == 0).astype(jnp.float32),
            ((lane == 1) | (lane == 2)).astype(jnp.float32),
        ]
    )
    xg = jnp.transpose(x[:, :, 0].reshape(B, T // _U, _U), (1, 0, 2))

    def body(x_ref, w_ref, misc_ref, hist_ref, sfin_ref, h_s, c_s):
        j = pl.program_id(0)
        e0 = misc_ref[0:1, :]
        e1 = misc_ref[1:2, :]

        @pl.when(j == 0)
        def _():
            h_s[...] = jnp.zeros_like(h_s)
            c_s[...] = jnp.zeros_like(c_s)

        Wv = w_ref[...].astype(jnp.bfloat16)
        pltpu.matmul_push_rhs(Wv[:, : 2 * H], 0, 0)
        pltpu.matmul_push_rhs(Wv[:, 2 * H :], 0, 1)
        zlat = jnp.zeros((8, 2 * H), jnp.float32)
        pltpu.matmul_acc_lhs(32, zlat, 0, load_staged_rhs=0)
        pltpu.matmul_acc_lhs(32, zlat, 1, load_staged_rhs=0)

        def group(gi, carry):
            hs, cs = carry
            t8 = gi * _U
            xs_grp = x_ref[pl.ds(gi, 1), :, :].reshape(B, _U)
            xpads = [xs_grp[:, u : u + 1] * e0 + e1 for u in range(_U)]
            hlists = [[] for _ in range(_C)]
            for u in range(_U):
                xpad = xpads[u]
                for cc in range(_C):
                    b0 = cc * BC
                    haug = jnp.concatenate([hs[cc], xpad[b0 : b0 + BC, :]], axis=1)
                    pltpu.matmul_acc_lhs(cc * 4, haug, 0, load_staged_rhs=None)
                    pltpu.matmul_acc_lhs(cc * 4, haug, 1, load_staged_rhs=None)
                new_hs, new_cs = [], []
                for cc in range(_C):
                    glo = pltpu.matmul_pop(cc * 4, (BC, 2 * H), jnp.float32, 0)
                    ghi = pltpu.matmul_pop(cc * 4, (BC, 2 * H), jnp.float32, 1)
                    i_g = _sigmoid(glo[:, :H])
                    f_g = _sigmoid(glo[:, H:])
                    g_g = jnp.tanh(ghi[:, :H])
                    o_g = _sigmoid(ghi[:, H:])
                    c_new = f_g * cs[cc] + i_g * g_g
                    h_new = o_g * jnp.tanh(c_new)
                    h = 0.5 * (hs[cc] + h_new)
                    c = 0.5 * (cs[cc] + c_new)
                    hlists[cc].append(h)
                    new_hs.append(h)
                    new_cs.append(c)
                hs, cs = tuple(new_hs), tuple(new_cs)
            tt0 = pl.multiple_of(t8, _U)
            for cc in range(_C):
                b0 = cc * BC
                hist_ref[b0 : b0 + BC, pl.ds(tt0, _U), :] = jnp.stack(
                    hlists[cc], axis=1
                )
            return (hs, cs)

        h0 = tuple(h_s[cc * BC : (cc + 1) * BC, :] for cc in range(_C))
        c0 = tuple(c_s[cc * BC : (cc + 1) * BC, :] for cc in range(_C))
        hs, cs = jax.lax.fori_loop(0, NG, group, (h0, c0))
        for cc in range(_C):
            h_s[cc * BC : (cc + 1) * BC, :] = hs[cc]
            c_s[cc * BC : (cc + 1) * BC, :] = cs[cc]

        @pl.when(j == NT - 1)
        def _():
            for cc in range(_C):
                b0 = cc * BC
                sfin_ref[b0 : b0 + BC, :H] = hs[cc]
                sfin_ref[b0 : b0 + BC, H:] = cs[cc]

    hist, sfin = pl.pallas_call(
        body,
        grid=(NT,),
        in_specs=[
            pl.BlockSpec((_TT // _U, B, _U), lambda j: (j, 0, 0)),
            pl.BlockSpec((2 * H, G4), lambda j: (0, 0)),
            pl.BlockSpec((2, H), lambda j: (0, 0)),
        ],
        out_specs=[
            pl.BlockSpec((B, _TT, H), lambda j: (0, j, 0)),
            pl.BlockSpec((B, 2 * H), lambda j: (0, 0)),
        ],
        out_shape=[
            jax.ShapeDtypeStruct((B, T, H), jnp.float32),
            jax.ShapeDtypeStruct((B, 2 * H), jnp.float32),
        ],
        scratch_shapes=[
            pltpu.VMEM((B, H), jnp.float32),
            pltpu.VMEM((B, H), jnp.float32),
        ],
        compiler_params=pltpu.CompilerParams(
            dimension_semantics=("arbitrary",),
        ),
    )(xg, Waug, misc)

    return hist, sfin
